# baseline (speedup 1.0000x reference)
import jax
import jax.numpy as jnp
import numpy as _np
from jax import lax
from jax.experimental import pallas as pl
from jax.experimental.pallas import tpu as pltpu
from jax.experimental.pallas import tpu_sc as plsc

B, N, M = 4, 4096, 1024
C1, C2 = 256, 512
CIN, H1, H2 = 768, 512, 512
BN_KNN = 1024
BN_MLP = 1024
MB = M // (N // BN_KNN)
HC2 = C2 // 2

NW = 32
PW = (B * N) // NW
CP = 32
NCHUNK = PW // CP

_PERM = _np.empty((C2,), dtype=_np.int32)
for _v in range(C2 // 32):
    for _t in range(16):
        _PERM[_v * 32 + _t] = _v * 16 + _t
        _PERM[_v * 32 + 16 + _t] = 256 + _v * 16 + _t


def _knn_body(u_ref, k_ref, kf_ref, i0_ref, i1_ref, i2_ref,
              w0_ref, w1_ref, w2_ref, kft_ref):
    b = pl.program_id(0)
    u = u_ref[0]
    k = k_ref[0]
    dx = k[:, 0:1] - u[0:1, :]
    dy = k[:, 1:2] - u[1:2, :]
    dz = k[:, 2:3] - u[2:3, :]
    d2 = dx * dx + dy * dy
    d2 = d2 + dz * dz

    iota_row = lax.broadcasted_iota(jnp.int32, (1, M), 1).astype(jnp.float32)
    dists, idxs = [], []
    cur = d2
    for j in range(3):
        mval = jnp.min(cur, axis=0)
        onehot = cur == mval[None, :]
        ohf = onehot.astype(jnp.float32)
        aidx = lax.dot_general(iota_row, ohf, (((1,), (0,)), ((), ())),
                               precision=lax.Precision.HIGHEST,
                               preferred_element_type=jnp.float32)
        dists.append(mval)
        idxs.append(jnp.minimum(aidx[0] + 0.5,
                                jnp.float32(M - 1)).astype(jnp.int32))
        if j < 2:
            cur = jnp.where(onehot, jnp.float32(jnp.inf), cur)

    r0 = 1.0 / (dists[0] + 1e-8)
    r1 = 1.0 / (dists[1] + 1e-8)
    r2 = 1.0 / (dists[2] + 1e-8)
    norm = r0 + r1 + r2

    base = b * M
    i0_ref[...] = idxs[0] + base
    i1_ref[...] = idxs[1] + base
    i2_ref[...] = idxs[2] + base
    w0_ref[...] = r0 / norm
    w1_ref[...] = r1 / norm
    w2_ref[...] = r2 / norm

    t = jnp.transpose(kf_ref[0], (1, 0))
    blo = lax.bitcast_convert_type(t[:, :HC2], jnp.int32)
    bhi = lax.bitcast_convert_type(t[:, HC2:], jnp.int32)
    rlo = blo + 0x7FFF + ((blo >> 16) & 1)
    rhi = bhi + 0x7FFF + ((bhi >> 16) & 1)
    kft_ref[...] = ((rlo >> 16) & 0xFFFF) | (rhi & jnp.int32(-65536))


def _knn(unknown_t, known, known_feats):
    grid = (B, N // BN_KNN)
    nb = N // BN_KNN
    flat = pl.BlockSpec((BN_KNN,), lambda b, n: (b * nb + n,))
    return pl.pallas_call(
        _knn_body,
        grid=grid,
        in_specs=[
            pl.BlockSpec((1, 3, BN_KNN), lambda b, n: (b, 0, n)),
            pl.BlockSpec((1, M, 3), lambda b, n: (b, 0, 0)),
            pl.BlockSpec((1, C2, MB), lambda b, n: (b, 0, n)),
        ],
        out_specs=[flat] * 6 + [
            pl.BlockSpec((MB, HC2), lambda b, n: (b * (M // MB) + n, 0)),
        ],
        out_shape=[jax.ShapeDtypeStruct((B * N,), jnp.int32)] * 3
        + [jax.ShapeDtypeStruct((B * N,), jnp.float32)] * 3
        + [jax.ShapeDtypeStruct((B * M, HC2), jnp.int32)],
    )(unknown_t, known, known_feats)


def _sc_interp_body(kft_hbm, i0_hbm, i1_hbm, i2_hbm, w0_hbm, w1_hbm, w2_hbm,
                    out_hbm, idx_v, w_v, rows0, rows1, acc0, acc1,
                    sem0, sem1, semo0, semo1):
    wid = lax.axis_index("s") * 2 + lax.axis_index("c")
    lo = wid * PW
    for j, (iref, wref) in enumerate(((i0_hbm, w0_hbm), (i1_hbm, w1_hbm),
                                      (i2_hbm, w2_hbm))):
        pltpu.sync_copy(iref.at[pl.ds(lo, PW)], idx_v.at[pl.ds(j * PW, PW)])
        pltpu.sync_copy(wref.at[pl.ds(lo, PW)], w_v.at[pl.ds(j * PW, PW)])

    def start_gather(c, buf_ref, sem):
        for j in range(3):
            idx_slice = idx_v.at[pl.ds(j * PW + c * CP, CP)]
            pltpu.async_copy(kft_hbm.at[idx_slice],
                             buf_ref.at[pl.ds(j * CP, CP)], sem)

    def wait_gather(c, buf_ref, sem):
        for j in range(3):
            idx_slice = idx_v.at[pl.ds(j * PW + c * CP, CP)]
            pltpu.make_async_copy(kft_hbm.at[idx_slice],
                                  buf_ref.at[pl.ds(j * CP, CP)], sem).wait()

    def out_slice(c):
        return out_hbm.at[pl.ds(wid * PW + c * CP, CP)]

    def combine(c, buf_ref, acc_ref):
        @plsc.parallel_loop(0, CP, unroll=4)
        def _(p):
            w0 = w_v[pl.ds(c * CP + p, 16)][0]
            w1 = w_v[pl.ds(PW + c * CP + p, 16)][0]
            w2 = w_v[pl.ds(2 * PW + c * CP + p, 16)][0]
            mk = jnp.int32(-65536)
            bc = lax.bitcast_convert_type
            for v in range(C2 // 32):
                sl = pl.ds(v * 16, 16)
                r0 = buf_ref[p, sl]
                r1 = buf_ref[CP + p, sl]
                r2 = buf_ref[2 * CP + p, sl]
                a0 = bc(r0 << 16, jnp.float32)
                b0 = bc(r0 & mk, jnp.float32)
                a1 = bc(r1 << 16, jnp.float32)
                b1 = bc(r1 & mk, jnp.float32)
                a2 = bc(r2 << 16, jnp.float32)
                b2 = bc(r2 & mk, jnp.float32)
                acc_ref[p, pl.ds(v * 32, 16)] = a0 * w0 + a1 * w1 + a2 * w2
                acc_ref[p, pl.ds(v * 32 + 16, 16)] = b0 * w0 + b1 * w1 + b2 * w2

    start_gather(0, rows0, sem0)

    @pl.loop(0, NCHUNK, step=2)
    def _(c):
        wait_gather(c, rows0, sem0)
        start_gather(c + 1, rows1, sem1)

        @pl.when(c >= 2)
        def _():
            pltpu.make_async_copy(acc0, out_slice(c - 2), semo0).wait()

        combine(c, rows0, acc0)
        pltpu.async_copy(acc0, out_slice(c), semo0)

        wait_gather(c + 1, rows1, sem1)

        @pl.when(c + 2 < NCHUNK)
        def _():
            start_gather(c + 2, rows0, sem0)

        @pl.when(c >= 2)
        def _():
            pltpu.make_async_copy(acc1, out_slice(c - 1), semo1).wait()

        combine(c + 1, rows1, acc1)
        pltpu.async_copy(acc1, out_slice(c + 1), semo1)

    pltpu.make_async_copy(acc0, out_slice(NCHUNK - 2), semo0).wait()
    pltpu.make_async_copy(acc1, out_slice(NCHUNK - 1), semo1).wait()


def _sc_interp(kft_i, i0, i1, i2, w0, w1, w2):
    mesh = plsc.VectorSubcoreMesh(core_axis_name="c", subcore_axis_name="s")
    f = pl.kernel(
        _sc_interp_body,
        out_type=jax.ShapeDtypeStruct((B * N, C2), jnp.float32),
        mesh=mesh,
        scratch_types=[
            pltpu.VMEM((3 * PW,), jnp.int32),
            pltpu.VMEM((3 * PW + 16,), jnp.float32),
            pltpu.VMEM((3 * CP, HC2), jnp.int32),
            pltpu.VMEM((3 * CP, HC2), jnp.int32),
            pltpu.VMEM((CP, C2), jnp.float32),
            pltpu.VMEM((CP, C2), jnp.float32),
            pltpu.SemaphoreType.DMA,
            pltpu.SemaphoreType.DMA,
            pltpu.SemaphoreType.DMA,
            pltpu.SemaphoreType.DMA,
        ],
    )
    return f(kft_i, i0, i1, i2, w0, w1, w2)


def _mlp_partial_body(xu_ref, w_ref, p_ref):
    p_ref[...] = lax.dot_general(xu_ref[0], w_ref[...],
                                 (((0,), (1,)), ((), ())),
                                 preferred_element_type=jnp.float32
                                 ).astype(jnp.bfloat16)


def _mlp_partial(unknow_feats, w0b):
    grid = (B, N // BN_MLP)
    nb = N // BN_MLP
    return pl.pallas_call(
        _mlp_partial_body,
        grid=grid,
        in_specs=[
            pl.BlockSpec((1, C1, BN_MLP), lambda b, n: (b, 0, n)),
            pl.BlockSpec((H1, C1), lambda b, n: (0, 0)),
        ],
        out_specs=pl.BlockSpec((BN_MLP, H1), lambda b, n: (b * nb + n, 0)),
        out_shape=jax.ShapeDtypeStruct((B * N, H1), jnp.bfloat16),
    )(unknow_feats, w0b)


_DN = (((1,), (1,)), ((), ()))


def _mlp1_body(xi_ref, p_ref, w_ref, y_ref, s_ref, ss_ref):
    first = pl.program_id(0) == 0
    y = lax.dot_general(xi_ref[...], w_ref[...], _DN,
                        preferred_element_type=jnp.float32
                        ) + p_ref[...].astype(jnp.float32)
    y_ref[...] = y.astype(jnp.bfloat16)

    @pl.when(first)
    def _():
        s_ref[...] = jnp.zeros_like(s_ref)
        ss_ref[...] = jnp.zeros_like(ss_ref)

    s_ref[...] += jnp.sum(y, axis=0, keepdims=True)
    ss_ref[...] += jnp.sum(y * y, axis=0, keepdims=True)


def _mlp1(interp, partial, w0a):
    grid = ((B * N) // BN_MLP,)
    return pl.pallas_call(
        _mlp1_body,
        grid=grid,
        in_specs=[
            pl.BlockSpec((BN_MLP, C2), lambda i: (i, 0)),
            pl.BlockSpec((BN_MLP, H1), lambda i: (i, 0)),
            pl.BlockSpec((H1, C2), lambda i: (0, 0)),
        ],
        out_specs=[
            pl.BlockSpec((BN_MLP, H1), lambda i: (i, 0)),
            pl.BlockSpec((1, H1), lambda i: (0, 0)),
            pl.BlockSpec((1, H1), lambda i: (0, 0)),
        ],
        out_shape=[
            jax.ShapeDtypeStruct((B * N, H1), jnp.bfloat16),
            jax.ShapeDtypeStruct((1, H1), jnp.float32),
            jax.ShapeDtypeStruct((1, H1), jnp.float32),
        ],
    )(interp, partial, w0a)


def _mlp2_body(y1_ref, s_ref, ss_ref, g_ref, b_ref, w_ref,
               y2_ref, s2_ref, ss2_ref):
    i = pl.program_id(0)
    nb = N // BN_MLP
    first = i == 0
    count = jnp.float32(B * N)
    mean = s_ref[...] / count
    var = ss_ref[...] / count - mean * mean
    scale = g_ref[...] / jnp.sqrt(var + 1e-5)
    shift = b_ref[...] - mean * scale

    h = jnp.maximum(y1_ref[...].astype(jnp.float32) * scale + shift, 0.0)
    y2 = lax.dot_general(h, w_ref[...], _DN, preferred_element_type=jnp.float32)
    y2_ref[0] = jnp.transpose(y2, (1, 0)).astype(jnp.bfloat16)

    @pl.when(first)
    def _():
        s2_ref[...] = jnp.zeros_like(s2_ref)
        ss2_ref[...] = jnp.zeros_like(ss2_ref)

    s2_ref[...] += jnp.sum(y2, axis=0, keepdims=True)
    ss2_ref[...] += jnp.sum(y2 * y2, axis=0, keepdims=True)


def _mlp2(y1, s, ss, g0r, b0r, W1):
    grid = ((B * N) // BN_MLP,)
    nb = N // BN_MLP
    return pl.pallas_call(
        _mlp2_body,
        grid=grid,
        in_specs=[
            pl.BlockSpec((BN_MLP, H1), lambda i: (i, 0)),
            pl.BlockSpec((1, H1), lambda i: (0, 0)),
            pl.BlockSpec((1, H1), lambda i: (0, 0)),
            pl.BlockSpec((1, H1), lambda i: (0, 0)),
            pl.BlockSpec((1, H1), lambda i: (0, 0)),
            pl.BlockSpec((H2, H1), lambda i: (0, 0)),
        ],
        out_specs=[
            pl.BlockSpec((1, H2, BN_MLP), lambda i: (i // nb, 0, i % nb)),
            pl.BlockSpec((1, H2), lambda i: (0, 0)),
            pl.BlockSpec((1, H2), lambda i: (0, 0)),
        ],
        out_shape=[
            jax.ShapeDtypeStruct((B, H2, N), jnp.bfloat16),
            jax.ShapeDtypeStruct((1, H2), jnp.float32),
            jax.ShapeDtypeStruct((1, H2), jnp.float32),
        ],
    )(y1, s, ss, g0r, b0r, W1)


def _final_body(y2_ref, s_ref, ss_ref, g_ref, b_ref, out_ref):
    count = jnp.float32(B * N)
    mean = jnp.transpose(s_ref[...], (1, 0)) / count
    var = jnp.transpose(ss_ref[...], (1, 0)) / count - mean * mean
    scale = g_ref[...] / jnp.sqrt(var + 1e-5)
    shift = b_ref[...] - mean * scale
    h = y2_ref[0].astype(jnp.float32) * scale + shift
    out_ref[0] = jnp.maximum(h, 0.0)


def _final(y2t, s2, ss2, g1c, b1c):
    grid = (B, N // BN_MLP)
    return pl.pallas_call(
        _final_body,
        grid=grid,
        in_specs=[
            pl.BlockSpec((1, H2, BN_MLP), lambda b, n: (b, 0, n)),
            pl.BlockSpec((1, H2), lambda b, n: (0, 0)),
            pl.BlockSpec((1, H2), lambda b, n: (0, 0)),
            pl.BlockSpec((H2, 1), lambda b, n: (0, 0)),
            pl.BlockSpec((H2, 1), lambda b, n: (0, 0)),
        ],
        out_specs=pl.BlockSpec((1, H2, BN_MLP), lambda b, n: (b, 0, n)),
        out_shape=jax.ShapeDtypeStruct((B, H2, N), jnp.float32),
    )(y2t, s2, ss2, g1c, b1c)


def kernel(unknown, known, unknow_feats, known_feats, W0, g0, b0, W1, g1, b1):
    unknown_t = jnp.transpose(unknown, (0, 2, 1))
    w0a = W0[:, :C2][:, _PERM]
    w0b = W0[:, C2:]

    i0, i1, i2, wt0, wt1, wt2, kft_i = _knn(unknown_t, known, known_feats)
    interp = _sc_interp(kft_i, i0, i1, i2, wt0, wt1, wt2)
    partial = _mlp_partial(unknow_feats, w0b)

    y1, s, ss = _mlp1(interp, partial, w0a)
    y2t, s2, ss2 = _mlp2(y1, s, ss, g0.reshape(1, H1), b0.reshape(1, H1), W1)
    return _final(y2t, s2, ss2, g1.reshape(H2, 1), b1.reshape(H2, 1))


# speedup vs baseline: 1.3825x; 1.3825x over previous
import jax
import jax.numpy as jnp
import numpy as _np
from jax import lax
from jax.experimental import pallas as pl
from jax.experimental.pallas import tpu as pltpu
from jax.experimental.pallas import tpu_sc as plsc

B, N, M = 4, 4096, 1024
C1, C2 = 256, 512
CIN, H1, H2 = 768, 512, 512
BN_KNN = 1024
BN_MLP = 1024
MB = M // (N // BN_KNN)
HC2 = C2 // 2

NW = 32
PW = (B * N) // NW
CP = 32
NCHUNK = PW // CP

_PERM = _np.empty((C2,), dtype=_np.int32)
for _v in range(C2 // 32):
    for _t in range(16):
        _PERM[_v * 32 + _t] = _v * 16 + _t
        _PERM[_v * 32 + 16 + _t] = 256 + _v * 16 + _t


def _knn_body(u_ref, k_ref, kf_ref, i0_ref, i1_ref, i2_ref,
              w0_ref, w1_ref, w2_ref, kft_ref):
    b = pl.program_id(0)
    u = u_ref[0]
    k = k_ref[0]
    dx = k[:, 0:1] - u[0:1, :]
    dy = k[:, 1:2] - u[1:2, :]
    dz = k[:, 2:3] - u[2:3, :]
    d2 = dx * dx + dy * dy
    d2 = d2 + dz * dz

    iota_i = lax.broadcasted_iota(jnp.int32, (2, M), 1)
    qr_rows = jnp.where(lax.broadcasted_iota(jnp.int32, (2, M), 0) == 0,
                        iota_i >> 6, iota_i & 63).astype(jnp.float32)
    dists, idxs = [], []
    cur = d2
    for j in range(3):
        mval = jnp.min(cur, axis=0)
        onehot = cur == mval[None, :]
        ohf = onehot.astype(jnp.float32)
        qr = lax.dot_general(qr_rows, ohf, (((1,), (0,)), ((), ())),
                             preferred_element_type=jnp.float32)
        aidx = qr[0] * 64.0 + qr[1]
        dists.append(mval)
        idxs.append(jnp.minimum(aidx + 0.5,
                                jnp.float32(M - 1)).astype(jnp.int32))
        if j < 2:
            cur = jnp.where(onehot, jnp.float32(jnp.inf), cur)

    r0 = 1.0 / (dists[0] + 1e-8)
    r1 = 1.0 / (dists[1] + 1e-8)
    r2 = 1.0 / (dists[2] + 1e-8)
    norm = r0 + r1 + r2

    base = b * M
    i0_ref[...] = idxs[0] + base
    i1_ref[...] = idxs[1] + base
    i2_ref[...] = idxs[2] + base
    w0_ref[...] = r0 / norm
    w1_ref[...] = r1 / norm
    w2_ref[...] = r2 / norm

    t = jnp.transpose(kf_ref[0], (1, 0))
    blo = lax.bitcast_convert_type(t[:, :HC2], jnp.int32)
    bhi = lax.bitcast_convert_type(t[:, HC2:], jnp.int32)
    rlo = blo + 0x7FFF + ((blo >> 16) & 1)
    rhi = bhi + 0x7FFF + ((bhi >> 16) & 1)
    kft_ref[...] = ((rlo >> 16) & 0xFFFF) | (rhi & jnp.int32(-65536))


def _knn(unknown_t, known, known_feats):
    grid = (B, N // BN_KNN)
    nb = N // BN_KNN
    flat = pl.BlockSpec((BN_KNN,), lambda b, n: (b * nb + n,))
    return pl.pallas_call(
        _knn_body,
        grid=grid,
        in_specs=[
            pl.BlockSpec((1, 3, BN_KNN), lambda b, n: (b, 0, n)),
            pl.BlockSpec((1, M, 3), lambda b, n: (b, 0, 0)),
            pl.BlockSpec((1, C2, MB), lambda b, n: (b, 0, n)),
        ],
        out_specs=[flat] * 6 + [
            pl.BlockSpec((MB, HC2), lambda b, n: (b * (M // MB) + n, 0)),
        ],
        out_shape=[jax.ShapeDtypeStruct((B * N,), jnp.int32)] * 3
        + [jax.ShapeDtypeStruct((B * N,), jnp.float32)] * 3
        + [jax.ShapeDtypeStruct((B * M, HC2), jnp.int32)],
    )(unknown_t, known, known_feats)


def _sc_interp_body(kft_hbm, i0_hbm, i1_hbm, i2_hbm, w0_hbm, w1_hbm, w2_hbm,
                    out_hbm, idx_v, w_v, rows0, rows1, acc0, acc1,
                    sem0, sem1, semo0, semo1):
    wid = lax.axis_index("s") * 2 + lax.axis_index("c")
    lo = wid * PW
    for j, (iref, wref) in enumerate(((i0_hbm, w0_hbm), (i1_hbm, w1_hbm),
                                      (i2_hbm, w2_hbm))):
        pltpu.sync_copy(iref.at[pl.ds(lo, PW)], idx_v.at[pl.ds(j * PW, PW)])
        pltpu.sync_copy(wref.at[pl.ds(lo, PW)], w_v.at[pl.ds(j * PW, PW)])

    def start_gather(c, buf_ref, sem):
        for j in range(3):
            idx_slice = idx_v.at[pl.ds(j * PW + c * CP, CP)]
            pltpu.async_copy(kft_hbm.at[idx_slice],
                             buf_ref.at[pl.ds(j * CP, CP)], sem)

    def wait_gather(c, buf_ref, sem):
        for j in range(3):
            idx_slice = idx_v.at[pl.ds(j * PW + c * CP, CP)]
            pltpu.make_async_copy(kft_hbm.at[idx_slice],
                                  buf_ref.at[pl.ds(j * CP, CP)], sem).wait()

    def out_slice(c):
        return out_hbm.at[pl.ds(wid * PW + c * CP, CP)]

    def combine(c, buf_ref, acc_ref):
        @plsc.parallel_loop(0, CP, unroll=4)
        def _(p):
            w0 = w_v[pl.ds(c * CP + p, 16)][0]
            w1 = w_v[pl.ds(PW + c * CP + p, 16)][0]
            w2 = w_v[pl.ds(2 * PW + c * CP + p, 16)][0]
            mk = jnp.int32(-65536)
            bc = lax.bitcast_convert_type
            for v in range(C2 // 32):
                sl = pl.ds(v * 16, 16)
                r0 = buf_ref[p, sl]
                r1 = buf_ref[CP + p, sl]
                r2 = buf_ref[2 * CP + p, sl]
                a0 = bc(r0 << 16, jnp.float32)
                b0 = bc(r0 & mk, jnp.float32)
                a1 = bc(r1 << 16, jnp.float32)
                b1 = bc(r1 & mk, jnp.float32)
                a2 = bc(r2 << 16, jnp.float32)
                b2 = bc(r2 & mk, jnp.float32)
                acc_ref[p, pl.ds(v * 32, 16)] = a0 * w0 + a1 * w1 + a2 * w2
                acc_ref[p, pl.ds(v * 32 + 16, 16)] = b0 * w0 + b1 * w1 + b2 * w2

    start_gather(0, rows0, sem0)

    @pl.loop(0, NCHUNK, step=2)
    def _(c):
        wait_gather(c, rows0, sem0)
        start_gather(c + 1, rows1, sem1)

        @pl.when(c >= 2)
        def _():
            pltpu.make_async_copy(acc0, out_slice(c - 2), semo0).wait()

        combine(c, rows0, acc0)
        pltpu.async_copy(acc0, out_slice(c), semo0)

        wait_gather(c + 1, rows1, sem1)

        @pl.when(c + 2 < NCHUNK)
        def _():
            start_gather(c + 2, rows0, sem0)

        @pl.when(c >= 2)
        def _():
            pltpu.make_async_copy(acc1, out_slice(c - 1), semo1).wait()

        combine(c + 1, rows1, acc1)
        pltpu.async_copy(acc1, out_slice(c + 1), semo1)

    pltpu.make_async_copy(acc0, out_slice(NCHUNK - 2), semo0).wait()
    pltpu.make_async_copy(acc1, out_slice(NCHUNK - 1), semo1).wait()


def _sc_interp(kft_i, i0, i1, i2, w0, w1, w2):
    mesh = plsc.VectorSubcoreMesh(core_axis_name="c", subcore_axis_name="s")
    f = pl.kernel(
        _sc_interp_body,
        out_type=jax.ShapeDtypeStruct((B * N, C2), jnp.float32),
        mesh=mesh,
        scratch_types=[
            pltpu.VMEM((3 * PW,), jnp.int32),
            pltpu.VMEM((3 * PW + 16,), jnp.float32),
            pltpu.VMEM((3 * CP, HC2), jnp.int32),
            pltpu.VMEM((3 * CP, HC2), jnp.int32),
            pltpu.VMEM((CP, C2), jnp.float32),
            pltpu.VMEM((CP, C2), jnp.float32),
            pltpu.SemaphoreType.DMA,
            pltpu.SemaphoreType.DMA,
            pltpu.SemaphoreType.DMA,
            pltpu.SemaphoreType.DMA,
        ],
    )
    return f(kft_i, i0, i1, i2, w0, w1, w2)


def _mlp_partial_body(xu_ref, w_ref, p_ref):
    p_ref[...] = lax.dot_general(xu_ref[0], w_ref[...],
                                 (((0,), (1,)), ((), ())),
                                 preferred_element_type=jnp.float32
                                 ).astype(jnp.bfloat16)


def _mlp_partial(unknow_feats, w0b):
    grid = (B, N // BN_MLP)
    nb = N // BN_MLP
    return pl.pallas_call(
        _mlp_partial_body,
        grid=grid,
        in_specs=[
            pl.BlockSpec((1, C1, BN_MLP), lambda b, n: (b, 0, n)),
            pl.BlockSpec((H1, C1), lambda b, n: (0, 0)),
        ],
        out_specs=pl.BlockSpec((BN_MLP, H1), lambda b, n: (b * nb + n, 0)),
        out_shape=jax.ShapeDtypeStruct((B * N, H1), jnp.bfloat16),
    )(unknow_feats, w0b)


_DN = (((1,), (1,)), ((), ()))


def _mlp1_body(xi_ref, p_ref, w_ref, y_ref, s_ref, ss_ref):
    first = pl.program_id(0) == 0
    y = lax.dot_general(xi_ref[...], w_ref[...], _DN,
                        preferred_element_type=jnp.float32
                        ) + p_ref[...].astype(jnp.float32)
    y_ref[...] = y.astype(jnp.bfloat16)

    @pl.when(first)
    def _():
        s_ref[...] = jnp.zeros_like(s_ref)
        ss_ref[...] = jnp.zeros_like(ss_ref)

    s_ref[...] += jnp.sum(y, axis=0, keepdims=True)
    ss_ref[...] += jnp.sum(y * y, axis=0, keepdims=True)


def _mlp1(interp, partial, w0a):
    grid = ((B * N) // BN_MLP,)
    return pl.pallas_call(
        _mlp1_body,
        grid=grid,
        in_specs=[
            pl.BlockSpec((BN_MLP, C2), lambda i: (i, 0)),
            pl.BlockSpec((BN_MLP, H1), lambda i: (i, 0)),
            pl.BlockSpec((H1, C2), lambda i: (0, 0)),
        ],
        out_specs=[
            pl.BlockSpec((BN_MLP, H1), lambda i: (i, 0)),
            pl.BlockSpec((1, H1), lambda i: (0, 0)),
            pl.BlockSpec((1, H1), lambda i: (0, 0)),
        ],
        out_shape=[
            jax.ShapeDtypeStruct((B * N, H1), jnp.bfloat16),
            jax.ShapeDtypeStruct((1, H1), jnp.float32),
            jax.ShapeDtypeStruct((1, H1), jnp.float32),
        ],
    )(interp, partial, w0a)


def _mlp2_body(y1_ref, s_ref, ss_ref, g_ref, b_ref, w_ref,
               y2_ref, s2_ref, ss2_ref):
    i = pl.program_id(0)
    nb = N // BN_MLP
    first = i == 0
    count = jnp.float32(B * N)
    mean = s_ref[...] / count
    var = ss_ref[...] / count - mean * mean
    scale = g_ref[...] / jnp.sqrt(var + 1e-5)
    shift = b_ref[...] - mean * scale

    h = jnp.maximum(y1_ref[...].astype(jnp.float32) * scale + shift, 0.0)
    y2 = lax.dot_general(h, w_ref[...], _DN, preferred_element_type=jnp.float32)
    y2_ref[0] = jnp.transpose(y2, (1, 0)).astype(jnp.bfloat16)

    @pl.when(first)
    def _():
        s2_ref[...] = jnp.zeros_like(s2_ref)
        ss2_ref[...] = jnp.zeros_like(ss2_ref)

    s2_ref[...] += jnp.sum(y2, axis=0, keepdims=True)
    ss2_ref[...] += jnp.sum(y2 * y2, axis=0, keepdims=True)


def _mlp2(y1, s, ss, g0r, b0r, W1):
    grid = ((B * N) // BN_MLP,)
    nb = N // BN_MLP
    return pl.pallas_call(
        _mlp2_body,
        grid=grid,
        in_specs=[
            pl.BlockSpec((BN_MLP, H1), lambda i: (i, 0)),
            pl.BlockSpec((1, H1), lambda i: (0, 0)),
            pl.BlockSpec((1, H1), lambda i: (0, 0)),
            pl.BlockSpec((1, H1), lambda i: (0, 0)),
            pl.BlockSpec((1, H1), lambda i: (0, 0)),
            pl.BlockSpec((H2, H1), lambda i: (0, 0)),
        ],
        out_specs=[
            pl.BlockSpec((1, H2, BN_MLP), lambda i: (i // nb, 0, i % nb)),
            pl.BlockSpec((1, H2), lambda i: (0, 0)),
            pl.BlockSpec((1, H2), lambda i: (0, 0)),
        ],
        out_shape=[
            jax.ShapeDtypeStruct((B, H2, N), jnp.bfloat16),
            jax.ShapeDtypeStruct((1, H2), jnp.float32),
            jax.ShapeDtypeStruct((1, H2), jnp.float32),
        ],
    )(y1, s, ss, g0r, b0r, W1)


def _final_body(y2_ref, s_ref, ss_ref, g_ref, b_ref, out_ref):
    count = jnp.float32(B * N)
    mean = jnp.transpose(s_ref[...], (1, 0)) / count
    var = jnp.transpose(ss_ref[...], (1, 0)) / count - mean * mean
    scale = g_ref[...] / jnp.sqrt(var + 1e-5)
    shift = b_ref[...] - mean * scale
    h = y2_ref[0].astype(jnp.float32) * scale + shift
    out_ref[0] = jnp.maximum(h, 0.0)


def _final(y2t, s2, ss2, g1c, b1c):
    grid = (B, N // BN_MLP)
    return pl.pallas_call(
        _final_body,
        grid=grid,
        in_specs=[
            pl.BlockSpec((1, H2, BN_MLP), lambda b, n: (b, 0, n)),
            pl.BlockSpec((1, H2), lambda b, n: (0, 0)),
            pl.BlockSpec((1, H2), lambda b, n: (0, 0)),
            pl.BlockSpec((H2, 1), lambda b, n: (0, 0)),
            pl.BlockSpec((H2, 1), lambda b, n: (0, 0)),
        ],
        out_specs=pl.BlockSpec((1, H2, BN_MLP), lambda b, n: (b, 0, n)),
        out_shape=jax.ShapeDtypeStruct((B, H2, N), jnp.float32),
    )(y2t, s2, ss2, g1c, b1c)


def kernel(unknown, known, unknow_feats, known_feats, W0, g0, b0, W1, g1, b1):
    unknown_t = jnp.transpose(unknown, (0, 2, 1))
    w0a = W0[:, :C2][:, _PERM]
    w0b = W0[:, C2:]

    i0, i1, i2, wt0, wt1, wt2, kft_i = _knn(unknown_t, known, known_feats)
    interp = _sc_interp(kft_i, i0, i1, i2, wt0, wt1, wt2)
    partial = _mlp_partial(unknow_feats, w0b)

    y1, s, ss = _mlp1(interp, partial, w0a)
    y2t, s2, ss2 = _mlp2(y1, s, ss, g0.reshape(1, H1), b0.reshape(1, H1), W1)
    return _final(y2t, s2, ss2, g1.reshape(H2, 1), b1.reshape(H2, 1))
